# baseline (speedup 1.0000x reference)
import functools

import jax
import jax.numpy as jnp
from jax import lax
from jax.experimental import pallas as pl
from jax.experimental.pallas import tpu as pltpu
from jax.experimental.pallas import tpu_sc as plsc

NC = 2
NS = 16
NW = NC * NS
L = 16


def _make_scatter(n, d, e_pad):
    w_edges = e_pad // NW
    st = 1024
    ch = 128
    n_stages = w_edges // st
    n_pieces = st // ch
    rows_per_sub = n // NS
    mesh = plsc.VectorSubcoreMesh(core_axis_name="c", subcore_axis_name="s")

    @functools.partial(
        pl.kernel,
        out_type=jax.ShapeDtypeStruct((NC, n, d), jnp.float32),
        mesh=mesh,
        scratch_types=[
            pltpu.VMEM((8, 128), jnp.int32),
            pltpu.VMEM((8, 128), jnp.int32),
            pltpu.VMEM((st,), jnp.float32),
            pltpu.VMEM((ch, d // 2), jnp.int32),
            pltpu.VMEM((ch, d // 2), jnp.int32),
            pltpu.VMEM((ch, d), jnp.float32),
            pltpu.VMEM_SHARED((n, d), jnp.float32),
            pltpu.SemaphoreType.DMA,
            pltpu.SemaphoreType.DMA,
        ],
        compiler_params=pltpu.CompilerParams(use_tc_tiling_on_sc=False),
    )
    def scatter_kernel(featbf_hbm, feat_hbm, src_hbm, dst_hbm, w_hbm,
                       zeros_hbm, out_hbm,
                       src_v, dst_v, w_v, rows_a, rows_b, rows_f, acc_sh,
                       sem_g, sem_s):
        cid = lax.axis_index("c")
        sid = lax.axis_index("s")
        wid = cid * NS + sid

        rbase = pl.multiple_of(sid * rows_per_sub, 8)

        @pl.when(cid == 0)
        def _():
            pltpu.sync_copy(feat_hbm.at[pl.ds(rbase, rows_per_sub)],
                            acc_sh.at[pl.ds(rbase, rows_per_sub)])

        @pl.when(cid != 0)
        def _():
            pltpu.sync_copy(zeros_hbm.at[pl.ds(rbase, rows_per_sub)],
                            acc_sh.at[pl.ds(rbase, rows_per_sub)])

        with jax.named_scope("sc_init_barrier"):
            plsc.subcore_barrier()

        bufs = [rows_a, rows_b]

        def scale(rows_bf, woff):
            def scale_body(g, carry2):
                wv = w_v[pl.ds(woff + g * L, L)]
                for l in range(L):
                    ws = jnp.broadcast_to(wv[l], (L,))
                    r = g * L + l
                    for c in range(d // 32):
                        v32 = rows_bf[r, pl.ds(c * L, L)]
                        a = lax.bitcast_convert_type(v32 << 16, jnp.float32)
                        b = lax.bitcast_convert_type(
                            v32 & jnp.int32(-65536), jnp.float32)
                        rows_f[r, pl.ds(c * 32, L)] = a * ws
                        rows_f[r, pl.ds(c * 32 + L, L)] = b * ws
                return carry2
            lax.fori_loop(0, ch // L, scale_body, 0)

        def stage_body(k, carry):
            ebase = pl.multiple_of(wid * w_edges + k * st, st)
            irow = pl.multiple_of(wid * (w_edges // 128) + k * n_pieces,
                                  n_pieces)
            pltpu.sync_copy(src_hbm.at[pl.ds(irow, n_pieces)], src_v)
            pltpu.sync_copy(dst_hbm.at[pl.ds(irow, n_pieces)], dst_v)
            pltpu.sync_copy(w_hbm.at[pl.ds(ebase, st)], w_v)
            g_descs = [None] * n_pieces
            s_descs = [None] * n_pieces
            g_descs[0] = pltpu.async_copy(
                featbf_hbm.at[src_v.at[0]], bufs[0], sem_g)
            for j in range(n_pieces):
                g_descs[j].wait()
                if j >= 1:
                    s_descs[j - 1].wait()
                if j + 1 < n_pieces:
                    g_descs[j + 1] = pltpu.async_copy(
                        featbf_hbm.at[src_v.at[j + 1]],
                        bufs[(j + 1) % 2], sem_g)
                scale(bufs[j % 2], j * ch)
                s_descs[j] = pltpu.async_copy(
                    rows_f, acc_sh.at[dst_v.at[j]], sem_s, add=True)
            s_descs[n_pieces - 1].wait()
            return carry

        with jax.named_scope("sc_edges"):
            lax.fori_loop(0, n_stages, stage_body, 0)
        with jax.named_scope("sc_writeback"):
            plsc.subcore_barrier()
            pltpu.sync_copy(acc_sh.at[pl.ds(rbase, rows_per_sub)],
                            out_hbm.at[cid, pl.ds(rbase, rows_per_sub)])

    return scatter_kernel


def _make_gather(n, d, b):
    bw = b // NW
    mesh = plsc.VectorSubcoreMesh(core_axis_name="c", subcore_axis_name="s")

    @functools.partial(
        pl.kernel,
        out_type=(jax.ShapeDtypeStruct((b, d), jnp.float32),
                  jax.ShapeDtypeStruct((b, d), jnp.float32)),
        mesh=mesh,
        scratch_types=[
            pltpu.VMEM((bw,), jnp.int32),
            pltpu.VMEM((bw, d), jnp.float32),
            pltpu.VMEM((bw, d), jnp.float32),
            pltpu.VMEM((bw, d), jnp.float32),
            pltpu.SemaphoreType.DMA,
        ],
    )
    def gather_kernel(agg0_hbm, agg1_hbm, qidx_hbm, pos_hbm, neg_hbm,
                      a_hbm, pm_hbm, idx_v, g0_v, g1_v, o_v, sem):
        cid = lax.axis_index("c")
        sid = lax.axis_index("s")
        wid = cid * NS + sid
        base = pl.multiple_of(wid * bw, 8)

        def fetch2(src_idx_hbm):
            pltpu.sync_copy(src_idx_hbm.at[pl.ds(base, bw)], idx_v)
            d0 = pltpu.async_copy(agg0_hbm.at[idx_v], g0_v, sem)
            d1 = pltpu.async_copy(agg1_hbm.at[idx_v], g1_v, sem)
            d0.wait()
            d1.wait()

        def combine(r, carry):
            for c in range(d // L):
                col = pl.ds(c * L, L)
                o_v[r, col] = (g0_v[r, col] + g1_v[r, col]) * 0.5
            return carry

        fetch2(qidx_hbm)
        lax.fori_loop(0, bw, combine, 0)
        pltpu.sync_copy(o_v, a_hbm.at[pl.ds(base, bw)])

        fetch2(pos_hbm)
        lax.fori_loop(0, bw, combine, 0)
        fetch2(neg_hbm)

        def combine_neg(r, carry):
            for c in range(d // L):
                col = pl.ds(c * L, L)
                o_v[r, col] = o_v[r, col] - (g0_v[r, col] + g1_v[r, col]) * 0.5
            return carry

        lax.fori_loop(0, bw, combine_neg, 0)
        pltpu.sync_copy(o_v, pm_hbm.at[pl.ds(base, bw)])

    return gather_kernel


def _score_body(a_ref, pm_ref, out_ref):
    i = pl.program_id(0)

    @pl.when(i == 0)
    def _():
        out_ref[0, 0] = 0.0

    s = lax.dot_general(a_ref[...], pm_ref[...],
                        (((1,), (1,)), ((), ())),
                        preferred_element_type=jnp.float32)
    loss = jnp.maximum(-s, 0.0) + jnp.log1p(jnp.exp(-jnp.abs(s)))
    out_ref[0, 0] += jnp.sum(loss)


def _make_score(b, d, tile):
    grid = b // tile
    return pl.pallas_call(
        _score_body,
        grid=(grid,),
        in_specs=[
            pl.BlockSpec((tile, d), lambda i: (i, 0)),
            pl.BlockSpec((b, d), lambda i: (0, 0)),
        ],
        out_specs=pl.BlockSpec(memory_space=pltpu.SMEM),
        out_shape=jax.ShapeDtypeStruct((1, 1), jnp.float32),
    )


def kernel(item_feature, edge_index, edge_weight, idx, sample_pair):
    n, d = item_feature.shape
    e = edge_weight.shape[0]
    b = idx.shape[0]

    per_w = -(-e // (NW * 1024)) * 1024
    e_pad = per_w * NW
    pad = e_pad - e
    pad_idx = jnp.arange(pad, dtype=jnp.int32) % jnp.int32(n)
    src = jnp.concatenate([edge_index[0], pad_idx])
    dst = jnp.concatenate([edge_index[1], pad_idx])
    w = jnp.concatenate([edge_weight, jnp.zeros((pad,), jnp.float32)])
    src2d = src.reshape(-1, 128)
    dst2d = dst.reshape(-1, 128)

    n_pad = -(-n // (NS * 8)) * NS * 8
    feat_p = jnp.concatenate(
        [item_feature, jnp.zeros((n_pad - n, d), jnp.float32)])
    zeros_nd = jnp.zeros((n_pad, d), jnp.float32)
    feat_bf = (feat_p.astype(jnp.bfloat16)
               .reshape(n_pad, d // 32, 2, 16)
               .transpose(0, 1, 3, 2)
               .reshape(n_pad, d // 2, 2))
    feat_bf = lax.bitcast_convert_type(feat_bf, jnp.int32)

    ab = _make_scatter(n_pad, d, e_pad)(feat_bf, feat_p, src2d, dst2d, w,
                                        zeros_nd)
    a, pm = _make_gather(n, d, b)(ab[0], ab[1], idx,
                                  sample_pair[:, 0], sample_pair[:, 1])
    total = _make_score(b, d, 512)(a, pm)
    return total[0, 0] / float(b * b)


# speedup vs baseline: 1.9354x; 1.9354x over previous
import functools

import jax
import jax.numpy as jnp
from jax import lax
from jax.experimental import pallas as pl
from jax.experimental.pallas import tpu as pltpu
from jax.experimental.pallas import tpu_sc as plsc

NC = 2
NS = 16
NW = NC * NS
L = 16


def _make_scatter_gather(n, d, e_pad, b):
    w_edges = e_pad // NW
    st = 1024
    ch = 128
    n_stages = w_edges // st
    n_pieces = st // ch
    rows_per_sub = n // NS
    bq = b // NS
    mesh = plsc.VectorSubcoreMesh(core_axis_name="c", subcore_axis_name="s")

    @functools.partial(
        pl.kernel,
        out_type=(jax.ShapeDtypeStruct((NC, b, d), jnp.float32),
                  jax.ShapeDtypeStruct((NC, b, d), jnp.float32)),
        mesh=mesh,
        scratch_types=[
            pltpu.VMEM((8, 128), jnp.int32),
            pltpu.VMEM((8, 128), jnp.int32),
            pltpu.VMEM((st,), jnp.float32),
            pltpu.VMEM((bq,), jnp.int32),
            pltpu.VMEM((ch, d), jnp.float32),
            pltpu.VMEM((ch, d), jnp.float32),
            pltpu.VMEM_SHARED((n, d), jnp.float32),
            pltpu.SemaphoreType.DMA,
            pltpu.SemaphoreType.DMA,
        ],
    )
    def scatter_kernel(feat_hbm, src_hbm, dst_hbm, w_hbm, zeros_hbm,
                       qidx_hbm, pos_hbm, neg_hbm,
                       a_hbm, pm_hbm,
                       src_v, dst_v, w_v, idx_v, rows_a, rows_b, acc_sh,
                       sem_g, sem_s):
        cid = lax.axis_index("c")
        sid = lax.axis_index("s")
        wid = cid * NS + sid

        rbase = pl.multiple_of(sid * rows_per_sub, 8)

        @pl.when(cid == 0)
        def _():
            pltpu.sync_copy(feat_hbm.at[pl.ds(rbase, rows_per_sub)],
                            acc_sh.at[pl.ds(rbase, rows_per_sub)])

        @pl.when(cid != 0)
        def _():
            pltpu.sync_copy(zeros_hbm.at[pl.ds(rbase, rows_per_sub)],
                            acc_sh.at[pl.ds(rbase, rows_per_sub)])

        with jax.named_scope("sc_init_barrier"):
            plsc.subcore_barrier()

        bufs = [rows_a, rows_b]

        def scale(rows_v, woff):
            def scale_body(g, carry2):
                wv = w_v[pl.ds(woff + g * L, L)]
                for l in range(L):
                    ws = jnp.broadcast_to(wv[l], (L,))
                    r = g * L + l
                    for c in range(d // L):
                        col = pl.ds(c * L, L)
                        rows_v[r, col] = rows_v[r, col] * ws
                return carry2
            lax.fori_loop(0, ch // L, scale_body, 0)

        def stage_body(k, carry):
            ebase = pl.multiple_of(wid * w_edges + k * st, st)
            irow = pl.multiple_of(wid * (w_edges // 128) + k * n_pieces,
                                  n_pieces)
            pltpu.sync_copy(src_hbm.at[pl.ds(irow, n_pieces)], src_v)
            pltpu.sync_copy(dst_hbm.at[pl.ds(irow, n_pieces)], dst_v)
            pltpu.sync_copy(w_hbm.at[pl.ds(ebase, st)], w_v)
            g_descs = [None] * n_pieces
            s_descs = [None] * n_pieces
            g_descs[0] = pltpu.async_copy(
                feat_hbm.at[src_v.at[0]], bufs[0], sem_g)
            for j in range(n_pieces):
                buf = bufs[j % 2]
                g_descs[j].wait()
                if j >= 1:
                    s_descs[j - 1].wait()
                if j + 1 < n_pieces:
                    g_descs[j + 1] = pltpu.async_copy(
                        feat_hbm.at[src_v.at[j + 1]],
                        bufs[(j + 1) % 2], sem_g)
                scale(buf, j * ch)
                s_descs[j] = pltpu.async_copy(
                    buf, acc_sh.at[dst_v.at[j]], sem_s, add=True)
            s_descs[n_pieces - 1].wait()
            return carry

        with jax.named_scope("sc_edges"):
            lax.fori_loop(0, n_stages, stage_body, 0)

        with jax.named_scope("sc_tail"):
            plsc.subcore_barrier()
            qbase = pl.multiple_of(sid * bq, 8)

            pltpu.sync_copy(qidx_hbm.at[pl.ds(qbase, bq)], idx_v)
            pltpu.async_copy(acc_sh.at[idx_v], rows_a, sem_g).wait()
            pltpu.sync_copy(rows_a, a_hbm.at[cid, pl.ds(qbase, bq)])

            pltpu.sync_copy(pos_hbm.at[pl.ds(qbase, bq)], idx_v)
            pltpu.async_copy(acc_sh.at[idx_v], rows_a, sem_g).wait()
            pltpu.sync_copy(neg_hbm.at[pl.ds(qbase, bq)], idx_v)
            pltpu.async_copy(acc_sh.at[idx_v], rows_b, sem_g).wait()

            def diff_body(r, carry):
                for c in range(d // L):
                    col = pl.ds(c * L, L)
                    rows_a[r, col] = rows_a[r, col] - rows_b[r, col]
                return carry
            lax.fori_loop(0, bq, diff_body, 0)
            pltpu.sync_copy(rows_a, pm_hbm.at[cid, pl.ds(qbase, bq)])

    return scatter_kernel


def _score_body(a_ref, pm_ref, out_ref):
    i = pl.program_id(0)

    @pl.when(i == 0)
    def _():
        out_ref[0, 0] = 0.0

    a = a_ref[0] + a_ref[1]
    pm = pm_ref[0] + pm_ref[1]
    s = 0.25 * lax.dot_general(a, pm,
                               (((1,), (1,)), ((), ())),
                               preferred_element_type=jnp.float32)
    loss = jnp.maximum(-s, 0.0) + jnp.log1p(jnp.exp(-jnp.abs(s)))
    out_ref[0, 0] += jnp.sum(loss)


def _make_score(b, d, tile):
    grid = b // tile
    return pl.pallas_call(
        _score_body,
        grid=(grid,),
        in_specs=[
            pl.BlockSpec((NC, tile, d), lambda i: (0, i, 0)),
            pl.BlockSpec((NC, b, d), lambda i: (0, 0, 0)),
        ],
        out_specs=pl.BlockSpec(memory_space=pltpu.SMEM),
        out_shape=jax.ShapeDtypeStruct((1, 1), jnp.float32),
    )


def kernel(item_feature, edge_index, edge_weight, idx, sample_pair):
    n, d = item_feature.shape
    e = edge_weight.shape[0]
    b = idx.shape[0]

    per_w = -(-e // (NW * 1024)) * 1024
    e_pad = per_w * NW
    pad = e_pad - e
    pad_idx = jnp.arange(pad, dtype=jnp.int32) % jnp.int32(n)
    src = jnp.concatenate([edge_index[0], pad_idx])
    dst = jnp.concatenate([edge_index[1], pad_idx])
    w = jnp.concatenate([edge_weight, jnp.zeros((pad,), jnp.float32)])
    src2d = src.reshape(-1, 128)
    dst2d = dst.reshape(-1, 128)

    n_pad = -(-n // (NS * 8)) * NS * 8
    feat_p = jnp.concatenate(
        [item_feature, jnp.zeros((n_pad - n, d), jnp.float32)])
    zeros_nd = jnp.zeros((n_pad, d), jnp.float32)

    ap, pp = _make_scatter_gather(n_pad, d, e_pad, b)(
        feat_p, src2d, dst2d, w, zeros_nd,
        idx, sample_pair[:, 0], sample_pair[:, 1])
    total = _make_score(b, d, 512)(ap, pp)
    return total[0, 0] / float(b * b)


# speedup vs baseline: 2.0436x; 1.0559x over previous
import functools

import jax
import jax.numpy as jnp
from jax import lax
from jax.experimental import pallas as pl
from jax.experimental.pallas import tpu as pltpu
from jax.experimental.pallas import tpu_sc as plsc

NC = 2
NS = 16
NW = NC * NS
L = 16


def _make_scatter_gather(n, d, e_pad, b):
    w_edges = e_pad // NW
    st = 2048
    ch = 128
    n_stages = w_edges // st
    n_pieces = st // ch
    rows_per_sub = n // NS
    bq = b // NS
    mesh = plsc.VectorSubcoreMesh(core_axis_name="c", subcore_axis_name="s")

    @functools.partial(
        pl.kernel,
        out_type=(jax.ShapeDtypeStruct((NC, b, d), jnp.float32),
                  jax.ShapeDtypeStruct((NC, b, d), jnp.float32)),
        mesh=mesh,
        scratch_types=[
            pltpu.VMEM((st // 128, 128), jnp.int32),
            pltpu.VMEM((st // 128, 128), jnp.int32),
            pltpu.VMEM((st,), jnp.float32),
            pltpu.VMEM((ch, d), jnp.float32),
            pltpu.VMEM((ch, d), jnp.float32),
            pltpu.VMEM_SHARED((n, d), jnp.float32),
            pltpu.SemaphoreType.DMA,
            pltpu.SemaphoreType.DMA,
        ],
    )
    def scatter_kernel(feat_hbm, src_hbm, dst_hbm, w_hbm, zeros_hbm,
                       qidx_hbm, pos_hbm, neg_hbm,
                       a_hbm, pm_hbm,
                       src_v, dst_v, w_v, rows_a, rows_b, acc_sh,
                       sem_g, sem_s):
        cid = lax.axis_index("c")
        sid = lax.axis_index("s")
        wid = cid * NS + sid

        rbase = pl.multiple_of(sid * rows_per_sub, 8)

        @pl.when(cid == 0)
        def _():
            pltpu.sync_copy(feat_hbm.at[pl.ds(rbase, rows_per_sub)],
                            acc_sh.at[pl.ds(rbase, rows_per_sub)])

        @pl.when(cid != 0)
        def _():
            pltpu.sync_copy(zeros_hbm.at[pl.ds(rbase, rows_per_sub)],
                            acc_sh.at[pl.ds(rbase, rows_per_sub)])

        with jax.named_scope("sc_init_barrier"):
            plsc.subcore_barrier()

        bufs = [rows_a, rows_b]

        def scale(rows_v, woff):
            def scale_body(g, carry2):
                wv = w_v[pl.ds(woff + g * L, L)]
                for l in range(L):
                    ws = jnp.broadcast_to(wv[l], (L,))
                    r = g * L + l
                    for c in range(d // L):
                        col = pl.ds(c * L, L)
                        rows_v[r, col] = rows_v[r, col] * ws
                return carry2
            lax.fori_loop(0, ch // L, scale_body, 0)

        def stage_body(k, carry):
            ebase = pl.multiple_of(wid * w_edges + k * st, st)
            irow = pl.multiple_of(wid * (w_edges // 128) + k * n_pieces,
                                  n_pieces)
            pltpu.sync_copy(src_hbm.at[pl.ds(irow, n_pieces)], src_v)
            pltpu.sync_copy(dst_hbm.at[pl.ds(irow, n_pieces)], dst_v)
            pltpu.sync_copy(w_hbm.at[pl.ds(ebase, st)], w_v)
            g_descs = [None] * n_pieces
            s_descs = [None] * n_pieces
            g_descs[0] = pltpu.async_copy(
                feat_hbm.at[src_v.at[0]], bufs[0], sem_g)
            for j in range(n_pieces):
                buf = bufs[j % 2]
                g_descs[j].wait()
                if j >= 1:
                    s_descs[j - 1].wait()
                if j + 1 < n_pieces:
                    g_descs[j + 1] = pltpu.async_copy(
                        feat_hbm.at[src_v.at[j + 1]],
                        bufs[(j + 1) % 2], sem_g)
                scale(buf, j * ch)
                s_descs[j] = pltpu.async_copy(
                    buf, acc_sh.at[dst_v.at[j]], sem_s, add=True)
            s_descs[n_pieces - 1].wait()
            return carry

        with jax.named_scope("sc_edges"):
            lax.fori_loop(0, n_stages, stage_body, 0)

        with jax.named_scope("sc_tail"):
            plsc.subcore_barrier()
            qbase = pl.multiple_of(sid * bq, 8)

            idx_row = src_v.at[0]
            pltpu.sync_copy(qidx_hbm.at[pl.ds(qbase, bq)], idx_row)
            pltpu.async_copy(acc_sh.at[idx_row], rows_a, sem_g).wait()
            pltpu.sync_copy(rows_a, a_hbm.at[cid, pl.ds(qbase, bq)])

            pltpu.sync_copy(pos_hbm.at[pl.ds(qbase, bq)], idx_row)
            pltpu.async_copy(acc_sh.at[idx_row], rows_a, sem_g).wait()
            pltpu.sync_copy(neg_hbm.at[pl.ds(qbase, bq)], idx_row)
            pltpu.async_copy(acc_sh.at[idx_row], rows_b, sem_g).wait()

            def diff_body(r, carry):
                for c in range(d // L):
                    col = pl.ds(c * L, L)
                    rows_a[r, col] = rows_a[r, col] - rows_b[r, col]
                return carry
            lax.fori_loop(0, bq, diff_body, 0)
            pltpu.sync_copy(rows_a, pm_hbm.at[cid, pl.ds(qbase, bq)])

    return scatter_kernel


def _score_body(a_ref, pm_ref, out_ref):
    i = pl.program_id(0)

    @pl.when(i == 0)
    def _():
        out_ref[0, 0] = 0.0

    a = a_ref[0] + a_ref[1]
    pm = pm_ref[0] + pm_ref[1]
    s = 0.25 * lax.dot_general(a, pm,
                               (((1,), (1,)), ((), ())),
                               preferred_element_type=jnp.float32)
    loss = jnp.maximum(-s, 0.0) + jnp.log1p(jnp.exp(-jnp.abs(s)))
    out_ref[0, 0] += jnp.sum(loss)


def _make_score(b, d, tile):
    grid = b // tile
    return pl.pallas_call(
        _score_body,
        grid=(grid,),
        in_specs=[
            pl.BlockSpec((NC, tile, d), lambda i: (0, i, 0)),
            pl.BlockSpec((NC, b, d), lambda i: (0, 0, 0)),
        ],
        out_specs=pl.BlockSpec(memory_space=pltpu.SMEM),
        out_shape=jax.ShapeDtypeStruct((1, 1), jnp.float32),
    )


def kernel(item_feature, edge_index, edge_weight, idx, sample_pair):
    n, d = item_feature.shape
    e = edge_weight.shape[0]
    b = idx.shape[0]

    per_w = -(-e // (NW * 2048)) * 2048
    e_pad = per_w * NW
    pad = e_pad - e
    pad_idx = jnp.arange(pad, dtype=jnp.int32) % jnp.int32(n)
    src = jnp.concatenate([edge_index[0], pad_idx])
    dst = jnp.concatenate([edge_index[1], pad_idx])
    w = jnp.concatenate([edge_weight, jnp.zeros((pad,), jnp.float32)])
    src2d = src.reshape(-1, 128)
    dst2d = dst.reshape(-1, 128)

    n_pad = -(-n // (NS * 8)) * NS * 8
    feat_p = jnp.concatenate(
        [item_feature, jnp.zeros((n_pad - n, d), jnp.float32)])
    zeros_nd = jnp.zeros((n_pad, d), jnp.float32)

    ap, pp = _make_scatter_gather(n_pad, d, e_pad, b)(
        feat_p, src2d, dst2d, w, zeros_nd,
        idx, sample_pair[:, 0], sample_pair[:, 1])
    total = _make_score(b, d, 512)(ap, pp)
    return total[0, 0] / float(b * b)


# speedup vs baseline: 2.0648x; 1.0104x over previous
import functools

import jax
import jax.numpy as jnp
from jax import lax
from jax.experimental import pallas as pl
from jax.experimental.pallas import tpu as pltpu
from jax.experimental.pallas import tpu_sc as plsc

NC = 2
NS = 16
NW = NC * NS
L = 16


def _make_scatter_gather(n, d, e_pad, b):
    w_edges = e_pad // NW
    st = 2048
    ch = 128
    n_stages = w_edges // st
    n_pieces = st // ch
    rows_per_sub = n // NS
    bq = b // NS
    mesh = plsc.VectorSubcoreMesh(core_axis_name="c", subcore_axis_name="s")

    @functools.partial(
        pl.kernel,
        out_type=(jax.ShapeDtypeStruct((NC, b, d), jnp.float32),
                  jax.ShapeDtypeStruct((NC, b, d), jnp.float32)),
        mesh=mesh,
        scratch_types=[
            pltpu.VMEM((st // 128, 128), jnp.int32),
            pltpu.VMEM((st // 128, 128), jnp.int32),
            pltpu.VMEM((st,), jnp.float32),
            pltpu.VMEM((ch, d), jnp.float32),
            pltpu.VMEM((ch, d), jnp.float32),
            pltpu.VMEM_SHARED((n, d), jnp.float32),
            pltpu.SemaphoreType.DMA,
            pltpu.SemaphoreType.DMA,
        ],
    )
    def scatter_kernel(feat_hbm, src_hbm, dst_hbm, w_hbm, zeros_hbm,
                       qidx_hbm, pos_hbm, neg_hbm,
                       a_hbm, pm_hbm,
                       src_v, dst_v, w_v, rows_a, rows_b, acc_sh,
                       sem_g, sem_s):
        cid = lax.axis_index("c")
        sid = lax.axis_index("s")
        wid = cid * NS + sid

        rbase = pl.multiple_of(sid * rows_per_sub, 8)

        @pl.when(cid == 0)
        def _():
            pltpu.sync_copy(feat_hbm.at[pl.ds(rbase, rows_per_sub)],
                            acc_sh.at[pl.ds(rbase, rows_per_sub)])

        @pl.when(cid != 0)
        def _():
            pltpu.sync_copy(zeros_hbm.at[pl.ds(rbase, rows_per_sub)],
                            acc_sh.at[pl.ds(rbase, rows_per_sub)])

        with jax.named_scope("sc_init_barrier"):
            plsc.subcore_barrier()

        bufs = [rows_a, rows_b]

        def scale(rows_v, woff):
            def scale_body(g, carry2):
                wv = w_v[pl.ds(woff + g * L, L)]
                for l in range(L):
                    ws = jnp.broadcast_to(wv[l], (L,))
                    r = g * L + l
                    for c in range(d // L):
                        col = pl.ds(c * L, L)
                        rows_v[r, col] = rows_v[r, col] * ws
                return carry2
            lax.fori_loop(0, ch // L, scale_body, 0)

        def stage_body(k, carry):
            ebase = pl.multiple_of(wid * w_edges + k * st, st)
            irow = pl.multiple_of(wid * (w_edges // 128) + k * n_pieces,
                                  n_pieces)
            pltpu.sync_copy(src_hbm.at[pl.ds(irow, n_pieces)], src_v)
            pltpu.sync_copy(dst_hbm.at[pl.ds(irow, n_pieces)], dst_v)
            pltpu.sync_copy(w_hbm.at[pl.ds(ebase, st)], w_v)
            g_descs = [None] * n_pieces
            s_descs = [None] * n_pieces
            g_descs[0] = pltpu.async_copy(
                feat_hbm.at[src_v.at[0]], bufs[0], sem_g)
            for j in range(n_pieces):
                buf = bufs[j % 2]
                g_descs[j].wait()
                if j >= 1:
                    s_descs[j - 1].wait()
                if j + 1 < n_pieces:
                    g_descs[j + 1] = pltpu.async_copy(
                        feat_hbm.at[src_v.at[j + 1]],
                        bufs[(j + 1) % 2], sem_g)
                scale(buf, j * ch)
                s_descs[j] = pltpu.async_copy(
                    buf, acc_sh.at[dst_v.at[j]], sem_s, add=True)
            s_descs[n_pieces - 1].wait()
            return carry

        with jax.named_scope("sc_edges"):
            lax.fori_loop(0, n_stages, stage_body, 0)

        with jax.named_scope("sc_tail"):
            plsc.subcore_barrier()
            qbase = pl.multiple_of(sid * bq, 8)

            idx_row = src_v.at[0]
            pltpu.sync_copy(qidx_hbm.at[pl.ds(qbase, bq)], idx_row)
            pltpu.async_copy(acc_sh.at[idx_row], rows_a, sem_g).wait()
            pltpu.sync_copy(rows_a, a_hbm.at[cid, pl.ds(qbase, bq)])

            pltpu.sync_copy(pos_hbm.at[pl.ds(qbase, bq)], idx_row)
            pltpu.async_copy(acc_sh.at[idx_row], rows_a, sem_g).wait()
            pltpu.sync_copy(neg_hbm.at[pl.ds(qbase, bq)], idx_row)
            pltpu.async_copy(acc_sh.at[idx_row], rows_b, sem_g).wait()

            def diff_body(r, carry):
                for c in range(d // L):
                    col = pl.ds(c * L, L)
                    rows_a[r, col] = rows_a[r, col] - rows_b[r, col]
                return carry
            lax.fori_loop(0, bq, diff_body, 0)
            pltpu.sync_copy(rows_a, pm_hbm.at[cid, pl.ds(qbase, bq)])

    return scatter_kernel


def _score_body(a_ref, pm_ref, out_ref):
    i = pl.program_id(0)

    @pl.when(i == 0)
    def _():
        out_ref[0, 0] = 0.0

    a = a_ref[0] + a_ref[1]
    pm = pm_ref[0] + pm_ref[1]
    s = 0.25 * lax.dot_general(a, pm,
                               (((1,), (1,)), ((), ())),
                               preferred_element_type=jnp.float32)
    loss = jnp.maximum(-s, 0.0) + jnp.log1p(jnp.exp(-jnp.abs(s)))
    out_ref[0, 0] += jnp.sum(loss)


def _make_score(b, d, tile):
    grid = b // tile
    return pl.pallas_call(
        _score_body,
        grid=(grid,),
        in_specs=[
            pl.BlockSpec((NC, tile, d), lambda i: (0, i, 0)),
            pl.BlockSpec((NC, b, d), lambda i: (0, 0, 0)),
        ],
        out_specs=pl.BlockSpec(memory_space=pltpu.SMEM),
        out_shape=jax.ShapeDtypeStruct((1, 1), jnp.float32),
    )


def kernel(item_feature, edge_index, edge_weight, idx, sample_pair):
    n, d = item_feature.shape
    e = edge_weight.shape[0]
    b = idx.shape[0]

    per_w = -(-e // (NW * 2048)) * 2048
    e_pad = per_w * NW
    pad = e_pad - e
    pad_idx = jnp.arange(pad, dtype=jnp.int32) % jnp.int32(n)
    src = jnp.concatenate([edge_index[0], pad_idx])
    dst = jnp.concatenate([edge_index[1], pad_idx])
    w = jnp.concatenate([edge_weight, jnp.zeros((pad,), jnp.float32)])
    src2d = src.reshape(-1, 128)
    dst2d = dst.reshape(-1, 128)

    n_pad = -(-n // (NS * 8)) * NS * 8
    feat_p = jnp.concatenate(
        [item_feature, jnp.zeros((n_pad - n, d), jnp.float32)])
    zeros_nd = jnp.zeros((n_pad, d), jnp.float32)

    ap, pp = _make_scatter_gather(n_pad, d, e_pad, b)(
        feat_p, src2d, dst2d, w, zeros_nd,
        idx, sample_pair[:, 0], sample_pair[:, 1])
    total = _make_score(b, d, b)(ap, pp)
    return total[0, 0] / float(b * b)


# speedup vs baseline: 2.1159x; 1.0247x over previous
import functools

import jax
import jax.numpy as jnp
from jax import lax
from jax.experimental import pallas as pl
from jax.experimental.pallas import tpu as pltpu
from jax.experimental.pallas import tpu_sc as plsc

NC = 2
NS = 16
NW = NC * NS
L = 16


def _make_scatter_gather(n, d, e_pad, b, n_real):
    w_edges = e_pad // NW
    st = 2048
    ch = 128
    n_stages = w_edges // st
    n_pieces = st // ch
    rows_per_sub = n // NS
    bq = b // NS
    last_full = n_real - (NS - 1) * rows_per_sub
    assert 0 < last_full <= rows_per_sub and last_full % 8 == 0
    mesh = plsc.VectorSubcoreMesh(core_axis_name="c", subcore_axis_name="s")

    @functools.partial(
        pl.kernel,
        out_type=(jax.ShapeDtypeStruct((NC, b, d), jnp.float32),
                  jax.ShapeDtypeStruct((NC, b, d), jnp.float32)),
        mesh=mesh,
        scratch_types=[
            pltpu.VMEM((st // 128, 128), jnp.int32),
            pltpu.VMEM((st // 128, 128), jnp.int32),
            pltpu.VMEM((st,), jnp.float32),
            pltpu.VMEM((ch, d), jnp.float32),
            pltpu.VMEM((ch, d), jnp.float32),
            pltpu.VMEM_SHARED((n, d), jnp.float32),
            pltpu.SemaphoreType.DMA,
            pltpu.SemaphoreType.DMA,
        ],
    )
    def scatter_kernel(feat_hbm, src_hbm, dst_hbm, w_hbm,
                       qidx_hbm, pos_hbm, neg_hbm,
                       a_hbm, pm_hbm,
                       src_v, dst_v, w_v, rows_a, rows_b, acc_sh,
                       sem_g, sem_s):
        cid = lax.axis_index("c")
        sid = lax.axis_index("s")
        wid = cid * NS + sid

        rbase = pl.multiple_of(sid * rows_per_sub, 8)

        @pl.when((cid == 0) & (sid < NS - 1))
        def _():
            pltpu.sync_copy(feat_hbm.at[pl.ds(rbase, rows_per_sub)],
                            acc_sh.at[pl.ds(rbase, rows_per_sub)])

        @pl.when((cid == 0) & (sid == NS - 1))
        def _():
            pltpu.sync_copy(feat_hbm.at[pl.ds(rbase, last_full)],
                            acc_sh.at[pl.ds(rbase, last_full)])

        @pl.when(cid != 0)
        def _():
            def zero_body(r, carry):
                for c in range(d // L):
                    rows_a[r, pl.ds(c * L, L)] = jnp.zeros((L,), jnp.float32)
                return carry
            lax.fori_loop(0, ch, zero_body, 0)
            nfull = rows_per_sub // ch
            for j in range(nfull):
                pltpu.sync_copy(rows_a,
                                acc_sh.at[pl.ds(rbase + j * ch, ch)])
            rem = rows_per_sub - nfull * ch
            if rem:
                pltpu.sync_copy(
                    rows_a.at[pl.ds(0, rem)],
                    acc_sh.at[pl.ds(rbase + nfull * ch, rem)])

        with jax.named_scope("sc_init_barrier"):
            plsc.subcore_barrier()

        bufs = [rows_a, rows_b]

        def scale(rows_v, woff):
            def scale_body(g, carry2):
                wv = w_v[pl.ds(woff + g * L, L)]
                for l in range(L):
                    ws = jnp.broadcast_to(wv[l], (L,))
                    r = g * L + l
                    for c in range(d // L):
                        col = pl.ds(c * L, L)
                        rows_v[r, col] = rows_v[r, col] * ws
                return carry2
            lax.fori_loop(0, ch // L, scale_body, 0)

        def stage_body(k, carry):
            ebase = pl.multiple_of(wid * w_edges + k * st, st)
            irow = pl.multiple_of(wid * (w_edges // 128) + k * n_pieces,
                                  n_pieces)
            pltpu.sync_copy(src_hbm.at[pl.ds(irow, n_pieces)], src_v)
            pltpu.sync_copy(dst_hbm.at[pl.ds(irow, n_pieces)], dst_v)
            pltpu.sync_copy(w_hbm.at[pl.ds(ebase, st)], w_v)
            g_descs = [None] * n_pieces
            s_descs = [None] * n_pieces
            g_descs[0] = pltpu.async_copy(
                feat_hbm.at[src_v.at[0]], bufs[0], sem_g)
            for j in range(n_pieces):
                buf = bufs[j % 2]
                g_descs[j].wait()
                if j >= 1:
                    s_descs[j - 1].wait()
                if j + 1 < n_pieces:
                    g_descs[j + 1] = pltpu.async_copy(
                        feat_hbm.at[src_v.at[j + 1]],
                        bufs[(j + 1) % 2], sem_g)
                scale(buf, j * ch)
                s_descs[j] = pltpu.async_copy(
                    buf, acc_sh.at[dst_v.at[j]], sem_s, add=True)
            s_descs[n_pieces - 1].wait()
            return carry

        with jax.named_scope("sc_edges"):
            lax.fori_loop(0, n_stages, stage_body, 0)

        with jax.named_scope("sc_tail"):
            plsc.subcore_barrier()
            qbase = pl.multiple_of(sid * bq, 8)

            idx_row = src_v.at[0]
            pltpu.sync_copy(qidx_hbm.at[pl.ds(qbase, bq)], idx_row)
            pltpu.async_copy(acc_sh.at[idx_row], rows_a, sem_g).wait()
            pltpu.sync_copy(rows_a, a_hbm.at[cid, pl.ds(qbase, bq)])

            pltpu.sync_copy(pos_hbm.at[pl.ds(qbase, bq)], idx_row)
            pltpu.async_copy(acc_sh.at[idx_row], rows_a, sem_g).wait()
            pltpu.sync_copy(neg_hbm.at[pl.ds(qbase, bq)], idx_row)
            pltpu.async_copy(acc_sh.at[idx_row], rows_b, sem_g).wait()

            def diff_body(r, carry):
                for c in range(d // L):
                    col = pl.ds(c * L, L)
                    rows_a[r, col] = rows_a[r, col] - rows_b[r, col]
                return carry
            lax.fori_loop(0, bq, diff_body, 0)
            pltpu.sync_copy(rows_a, pm_hbm.at[cid, pl.ds(qbase, bq)])

    return scatter_kernel


def _score_body(a_ref, pm_ref, out_ref):
    i = pl.program_id(0)

    @pl.when(i == 0)
    def _():
        out_ref[0, 0] = 0.0

    a = a_ref[0] + a_ref[1]
    pm = pm_ref[0] + pm_ref[1]
    s = 0.25 * lax.dot_general(a, pm,
                               (((1,), (1,)), ((), ())),
                               preferred_element_type=jnp.float32)
    loss = jnp.maximum(-s, 0.0) + jnp.log1p(jnp.exp(-jnp.abs(s)))
    out_ref[0, 0] += jnp.sum(loss)


def _make_score(b, d, tile):
    grid = b // tile
    return pl.pallas_call(
        _score_body,
        grid=(grid,),
        in_specs=[
            pl.BlockSpec((NC, tile, d), lambda i: (0, i, 0)),
            pl.BlockSpec((NC, b, d), lambda i: (0, 0, 0)),
        ],
        out_specs=pl.BlockSpec(memory_space=pltpu.SMEM),
        out_shape=jax.ShapeDtypeStruct((1, 1), jnp.float32),
    )


def kernel(item_feature, edge_index, edge_weight, idx, sample_pair):
    n, d = item_feature.shape
    e = edge_weight.shape[0]
    b = idx.shape[0]

    per_w = -(-e // (NW * 2048)) * 2048
    e_pad = per_w * NW
    pad = e_pad - e
    pad_idx = jnp.arange(pad, dtype=jnp.int32) % jnp.int32(n)
    src = jnp.concatenate([edge_index[0], pad_idx])
    dst = jnp.concatenate([edge_index[1], pad_idx])
    w = jnp.concatenate([edge_weight, jnp.zeros((pad,), jnp.float32)])
    src2d = src.reshape(-1, 128)
    dst2d = dst.reshape(-1, 128)

    n_pad = -(-n // (NS * 8)) * NS * 8

    ap, pp = _make_scatter_gather(n_pad, d, e_pad, b, n)(
        item_feature, src2d, dst2d, w,
        idx, sample_pair[:, 0], sample_pair[:, 1])
    total = _make_score(b, d, b)(ap, pp)
    return total[0, 0] / float(b * b)


# speedup vs baseline: 2.1745x; 1.0277x over previous
import functools

import jax
import jax.numpy as jnp
from jax import lax
from jax.experimental import pallas as pl
from jax.experimental.pallas import tpu as pltpu
from jax.experimental.pallas import tpu_sc as plsc

NC = 2
NS = 16
NW = NC * NS
L = 16


def _make_scatter_gather(n, d, e_pad, b, n_real):
    w_edges = e_pad // NW
    st = 2048
    ch = 128
    n_stages = w_edges // st
    n_pieces = st // ch
    rows_per_sub = n // NS
    bq = b // NS
    last_full = n_real - (NS - 1) * rows_per_sub
    assert 0 < last_full <= rows_per_sub and last_full % 8 == 0
    mesh = plsc.VectorSubcoreMesh(core_axis_name="c", subcore_axis_name="s")

    @functools.partial(
        pl.kernel,
        out_type=(jax.ShapeDtypeStruct((NC, b, d), jnp.float32),
                  jax.ShapeDtypeStruct((NC, b, d), jnp.float32)),
        mesh=mesh,
        scratch_types=[
            pltpu.VMEM((st // 128, 128), jnp.int32),
            pltpu.VMEM((st // 128, 128), jnp.int32),
            pltpu.VMEM((st,), jnp.float32),
            pltpu.VMEM((ch, d), jnp.float32),
            pltpu.VMEM((ch, d), jnp.float32),
            pltpu.VMEM_SHARED((n, d), jnp.float32),
            pltpu.SemaphoreType.DMA,
            pltpu.SemaphoreType.DMA,
            pltpu.SemaphoreType.DMA,
        ],
    )
    def scatter_kernel(feat_hbm, src_hbm, dst_hbm, w_hbm,
                       qidx_hbm, pos_hbm, neg_hbm,
                       a_hbm, pm_hbm,
                       src_v, dst_v, w_v, rows_a, rows_b, acc_sh,
                       sem_g, sem_s, sem_i):
        cid = lax.axis_index("c")
        sid = lax.axis_index("s")
        wid = cid * NS + sid

        rbase = pl.multiple_of(sid * rows_per_sub, 8)

        @pl.when((cid == 0) & (sid < NS - 1))
        def _():
            pltpu.sync_copy(feat_hbm.at[pl.ds(rbase, rows_per_sub)],
                            acc_sh.at[pl.ds(rbase, rows_per_sub)])

        @pl.when((cid == 0) & (sid == NS - 1))
        def _():
            pltpu.sync_copy(feat_hbm.at[pl.ds(rbase, last_full)],
                            acc_sh.at[pl.ds(rbase, last_full)])

        @pl.when(cid != 0)
        def _():
            def zero_body(r, carry):
                for c in range(d // L):
                    rows_a[r, pl.ds(c * L, L)] = jnp.zeros((L,), jnp.float32)
                return carry
            lax.fori_loop(0, ch, zero_body, 0)
            nfull = rows_per_sub // ch
            for j in range(nfull):
                pltpu.sync_copy(rows_a,
                                acc_sh.at[pl.ds(rbase + j * ch, ch)])
            rem = rows_per_sub - nfull * ch
            if rem:
                pltpu.sync_copy(
                    rows_a.at[pl.ds(0, rem)],
                    acc_sh.at[pl.ds(rbase + nfull * ch, rem)])

        with jax.named_scope("sc_init_barrier"):
            plsc.subcore_barrier()

        bufs = [rows_a, rows_b]

        def scale(rows_v, woff):
            def scale_body(g, carry2):
                wv = w_v[pl.ds(woff + g * L, L)]
                for l in range(L):
                    ws = jnp.broadcast_to(wv[l], (L,))
                    r = g * L + l
                    for c in range(d // L):
                        col = pl.ds(c * L, L)
                        rows_v[r, col] = rows_v[r, col] * ws
                return carry2
            lax.fori_loop(0, ch // L, scale_body, 0)

        def stage_idx(k):
            ebase = pl.multiple_of(wid * w_edges + k * st, st)
            irow = pl.multiple_of(wid * (w_edges // 128) + k * n_pieces,
                                  n_pieces)
            return ebase, irow

        def issue_staging(k):
            ebase, irow = stage_idx(k)
            pltpu.async_copy(src_hbm.at[pl.ds(irow, n_pieces)], src_v, sem_i)
            pltpu.async_copy(dst_hbm.at[pl.ds(irow, n_pieces)], dst_v, sem_i)
            pltpu.async_copy(w_hbm.at[pl.ds(ebase, st)], w_v, sem_i)

        def wait_staging(k):
            ebase, irow = stage_idx(k)
            pltpu.make_async_copy(
                src_hbm.at[pl.ds(irow, n_pieces)], src_v, sem_i).wait()
            pltpu.make_async_copy(
                dst_hbm.at[pl.ds(irow, n_pieces)], dst_v, sem_i).wait()
            pltpu.make_async_copy(
                w_hbm.at[pl.ds(ebase, st)], w_v, sem_i).wait()

        def stage_body(k, carry):
            wait_staging(k)
            g_descs = [None] * n_pieces
            s_descs = [None] * n_pieces
            g_descs[0] = pltpu.async_copy(
                feat_hbm.at[src_v.at[0]], bufs[0], sem_g)
            for j in range(n_pieces):
                buf = bufs[j % 2]
                g_descs[j].wait()
                if j >= 1:
                    s_descs[j - 1].wait()
                if j + 1 < n_pieces:
                    g_descs[j + 1] = pltpu.async_copy(
                        feat_hbm.at[src_v.at[j + 1]],
                        bufs[(j + 1) % 2], sem_g)
                scale(buf, j * ch)
                s_descs[j] = pltpu.async_copy(
                    buf, acc_sh.at[dst_v.at[j]], sem_s, add=True)
            s_descs[n_pieces - 1].wait()

            @pl.when(k < n_stages - 1)
            def _():
                issue_staging(k + 1)
            return carry

        with jax.named_scope("sc_edges"):
            issue_staging(0)
            lax.fori_loop(0, n_stages, stage_body, 0)

        with jax.named_scope("sc_tail"):
            plsc.subcore_barrier()
            qbase = pl.multiple_of(sid * bq, 8)

            idx_row = src_v.at[0]
            pltpu.sync_copy(qidx_hbm.at[pl.ds(qbase, bq)], idx_row)
            pltpu.async_copy(acc_sh.at[idx_row], rows_a, sem_g).wait()
            pltpu.sync_copy(rows_a, a_hbm.at[cid, pl.ds(qbase, bq)])

            pltpu.sync_copy(pos_hbm.at[pl.ds(qbase, bq)], idx_row)
            pltpu.async_copy(acc_sh.at[idx_row], rows_a, sem_g).wait()
            pltpu.sync_copy(neg_hbm.at[pl.ds(qbase, bq)], idx_row)
            pltpu.async_copy(acc_sh.at[idx_row], rows_b, sem_g).wait()

            def diff_body(r, carry):
                for c in range(d // L):
                    col = pl.ds(c * L, L)
                    rows_a[r, col] = rows_a[r, col] - rows_b[r, col]
                return carry
            lax.fori_loop(0, bq, diff_body, 0)
            pltpu.sync_copy(rows_a, pm_hbm.at[cid, pl.ds(qbase, bq)])

    return scatter_kernel


def _score_body(a_ref, pm_ref, out_ref):
    i = pl.program_id(0)

    @pl.when(i == 0)
    def _():
        out_ref[0, 0] = 0.0

    a = a_ref[0] + a_ref[1]
    pm = pm_ref[0] + pm_ref[1]
    s = 0.25 * lax.dot_general(a, pm,
                               (((1,), (1,)), ((), ())),
                               preferred_element_type=jnp.float32)
    loss = jnp.maximum(-s, 0.0) + jnp.log1p(jnp.exp(-jnp.abs(s)))
    out_ref[0, 0] += jnp.sum(loss)


def _make_score(b, d, tile):
    grid = b // tile
    return pl.pallas_call(
        _score_body,
        grid=(grid,),
        in_specs=[
            pl.BlockSpec((NC, tile, d), lambda i: (0, i, 0)),
            pl.BlockSpec((NC, b, d), lambda i: (0, 0, 0)),
        ],
        out_specs=pl.BlockSpec(memory_space=pltpu.SMEM),
        out_shape=jax.ShapeDtypeStruct((1, 1), jnp.float32),
    )


def kernel(item_feature, edge_index, edge_weight, idx, sample_pair):
    n, d = item_feature.shape
    e = edge_weight.shape[0]
    b = idx.shape[0]

    per_w = -(-e // (NW * 2048)) * 2048
    e_pad = per_w * NW
    pad = e_pad - e
    pad_idx = jnp.arange(pad, dtype=jnp.int32) % jnp.int32(n)
    src = jnp.concatenate([edge_index[0], pad_idx])
    dst = jnp.concatenate([edge_index[1], pad_idx])
    w = jnp.concatenate([edge_weight, jnp.zeros((pad,), jnp.float32)])
    src2d = src.reshape(-1, 128)
    dst2d = dst.reshape(-1, 128)

    n_pad = -(-n // (NS * 8)) * NS * 8

    ap, pp = _make_scatter_gather(n_pad, d, e_pad, b, n)(
        item_feature, src2d, dst2d, w,
        idx, sample_pair[:, 0], sample_pair[:, 1])
    total = _make_score(b, d, b)(ap, pp)
    return total[0, 0] / float(b * b)
